# baseline (speedup 1.0000x reference)
import functools

import jax
import jax.numpy as jnp
from jax import lax
from jax.experimental import pallas as pl
from jax.experimental.pallas import tpu as pltpu
from jax.experimental.pallas import tpu_sc as plsc

_LANES = 16
_NSLOTS = 4
_LEAD = 2
_HPAD = 56


def _gather_sum_sc(ids_flat, table2, n_items, hist, d):
    half, d2 = table2.shape
    info = plsc.get_sparse_core_info()
    nc, ns = info.num_cores, info.num_subcores
    nw = nc * ns
    ipw = n_items // nw
    nvec = d // _LANES
    half_items = ipw // 2
    stride = 2 * half_items + 8
    mesh = plsc.VectorSubcoreMesh(core_axis_name="c", subcore_axis_name="s")

    @functools.partial(
        pl.kernel,
        out_type=jax.ShapeDtypeStruct((n_items * d,), jnp.float32),
        mesh=mesh,
        scratch_types=[
            pltpu.VMEM((ipw * _HPAD,), jnp.int32),
            pltpu.VMEM((ipw, _HPAD), jnp.int32),
            pltpu.VMEM((ipw, _HPAD), jnp.int32),
            pltpu.VMEM((_NSLOTS, _HPAD, 2 * d), jnp.float32),
            pltpu.VMEM((ipw, 2 * d), jnp.float32),
            pltpu.VMEM((ipw * d,), jnp.float32),
            pltpu.VMEM_SHARED((ns * (ipw + 8), 2 * d), jnp.float32),
            pltpu.SemaphoreType.DMA,
            pltpu.SemaphoreType.DMA,
            pltpu.SemaphoreType.DMA,
            pltpu.SemaphoreType.DMA,
            pltpu.SemaphoreType.DMA,
            pltpu.SemaphoreType.DMA,
            pltpu.SemaphoreType.DMA,
            pltpu.SemaphoreType.DMA,
        ],
        compiler_params=pltpu.CompilerParams(use_tc_tiling_on_sc=True),
    )
    def body(ids_hbm, table_hbm, out_hbm, idx_v, q_v, tgt_v, buf, accv, out_v,
             acc_s, *sems):
        sem_g, sem_s = sems[:_NSLOTS], sems[_NSLOTS:]
        sid = lax.axis_index("s")
        wid = sid * nc + lax.axis_index("c")
        abase = sid * stride
        dump = abase + 2 * half_items
        pltpu.sync_copy(ids_hbm.at[pl.ds(wid * ipw * _HPAD, ipw * _HPAD)],
                        idx_v)

        lane = lax.iota(jnp.int32, 16)
        offs = (0, 16, 32, 40)

        def prep(i, c):
            for o in offs:
                raw = idx_v[pl.ds(i * _HPAD + o, _LANES)]
                q_v[i, pl.ds(o, _LANES)] = lax.shift_right_logical(raw, 1)
                t = (abase + 2 * lax.bitwise_and(i, half_items - 1)
                     + lax.bitwise_and(raw, 1))
                if o == 40:
                    t = jnp.where(lane < hist - o, t, dump)
                tgt_v[i, pl.ds(o, _LANES)] = t
            return c

        lax.fori_loop(0, ipw, prep, 0)

        zeros = jnp.zeros((_LANES,), jnp.float32)

        def zbody(i, c):
            for j in range(2 * nvec):
                buf[0, i, pl.ds(_LANES * j, _LANES)] = zeros
            return c

        for p in range(2):
            base = p * half_items
            lax.fori_loop(0, _HPAD, zbody, 0)
            for zo in range(0, stride, _HPAD):
                zn = min(_HPAD, stride - zo)
                pltpu.sync_copy(buf.at[0, pl.ds(0, zn)],
                                acc_s.at[pl.ds(abase + zo, zn)])
            for c in range(_LEAD):
                pltpu.async_copy(table_hbm.at[q_v.at[base + c]], buf.at[c],
                                 sem_g[c])

            def steps(kk, carry):
                for b in range(_NSLOTS):
                    k = base + kk * _NSLOTS + b
                    pltpu.make_async_copy(
                        table_hbm.at[q_v.at[k]], buf.at[b], sem_g[b]).wait()
                    pltpu.async_copy(
                        buf.at[b], acc_s.at[tgt_v.at[k]], sem_s[b], add=True)
                    g = k + _LEAD
                    bg = (b + _LEAD) % _NSLOTS

                    @pl.when(g < base + half_items)
                    def _():
                        @pl.when(g >= base + _NSLOTS)
                        def _():
                            pltpu.make_async_copy(
                                buf.at[bg], acc_s.at[tgt_v.at[k]],
                                sem_s[bg]).wait()

                        pltpu.async_copy(
                            table_hbm.at[q_v.at[g]], buf.at[bg], sem_g[bg])
                return carry

            lax.fori_loop(0, half_items // _NSLOTS, steps, 0)

            for b in range(_NSLOTS):
                pltpu.make_async_copy(
                    buf.at[b], acc_s.at[tgt_v.at[0]], sem_s[b]).wait()

            pltpu.sync_copy(acc_s.at[pl.ds(abase, 2 * half_items)], accv)

            def comb(i, c):
                for j in range(nvec):
                    lo = accv[2 * i, pl.ds(_LANES * j, _LANES)]
                    hi = accv[2 * i + 1, pl.ds(d + _LANES * j, _LANES)]
                    out_v[pl.ds((base + i) * d + _LANES * j, _LANES)] = lo + hi
                return c

            lax.fori_loop(0, half_items, comb, 0)

        pltpu.sync_copy(out_v, out_hbm.at[pl.ds(wid * ipw * d, ipw * d)])

    return body(ids_flat, table2)


def _mm_body(scale, r_ref, t_ref, o_ref):
    o_ref[...] = jnp.dot(
        r_ref[...], t_ref[...], preferred_element_type=jnp.float32) * scale


def _matmul_tc(ratio, sums, scale):
    b, n = ratio.shape
    _, d = sums.shape
    bb = 256
    return pl.pallas_call(
        functools.partial(_mm_body, scale),
        grid=(b // bb,),
        in_specs=[
            pl.BlockSpec((bb, n), lambda i: (i, 0)),
            pl.BlockSpec((n, d), lambda i: (0, 0)),
        ],
        out_specs=pl.BlockSpec((bb, d), lambda i: (i, 0)),
        out_shape=jax.ShapeDtypeStruct((b, d), jnp.float32),
    )(ratio, sums)


def kernel(input_ids, input_ratio, embedding):
    n_items, hist = input_ids.shape
    vocab, d = embedding.shape
    ids_flat = jnp.pad(input_ids.astype(jnp.int32),
                       ((0, 0), (0, _HPAD - hist))).reshape(-1)
    table2 = embedding.reshape(vocab // 2, 2 * d)
    sums = _gather_sum_sc(ids_flat, table2, n_items, hist, d)
    return _matmul_tc(input_ratio, sums.reshape(n_items, d),
                      float(1.0 / hist))


# speedup vs baseline: 4.5492x; 4.5492x over previous
import functools

import jax
import jax.numpy as jnp
from jax import lax
from jax.experimental import pallas as pl
from jax.experimental.pallas import tpu as pltpu
from jax.experimental.pallas import tpu_sc as plsc

_LANES = 16
_NSLOTS = 4
_LEAD = 2


def _detile_body(bcols, x_ref, o_ref):
    y = x_ref[...].T
    z = jnp.concatenate([y, jnp.zeros_like(y)], axis=1)
    o_ref[...] = jnp.reshape(z, (bcols * 2 * x_ref.shape[0],))


def _detile_tc(table_t):
    d, v = table_t.shape
    bcols = 8192
    return pl.pallas_call(
        functools.partial(_detile_body, bcols),
        grid=(pl.cdiv(v, bcols),),
        in_specs=[pl.BlockSpec((d, bcols), lambda i: (0, i))],
        out_specs=pl.BlockSpec((bcols * 2 * d,), lambda i: (i,)),
        out_shape=jax.ShapeDtypeStruct((v * 2 * d,), jnp.float32),
    )(table_t)


def _gather_sum_sc(ids, tgt, table):
    n_items, hist = ids.shape
    _, d = table.shape
    info = plsc.get_sparse_core_info()
    nc, ns = info.num_cores, info.num_subcores
    nw = nc * ns
    ipw = n_items // nw
    nvec = d // _LANES
    mesh = plsc.VectorSubcoreMesh(core_axis_name="c", subcore_axis_name="s")

    @functools.partial(
        pl.kernel,
        out_type=jax.ShapeDtypeStruct((n_items, d), jnp.float32),
        mesh=mesh,
        scratch_types=[
            pltpu.VMEM((ipw, hist), jnp.int32),
            pltpu.VMEM((ipw, hist), jnp.int32),
            pltpu.VMEM((ipw, hist), jnp.int32),
            pltpu.VMEM((_NSLOTS, hist, d), jnp.float32),
            pltpu.VMEM((ipw, d), jnp.float32),
            pltpu.VMEM_SHARED((ns * ipw, d), jnp.float32),
            pltpu.SemaphoreType.DMA,
            pltpu.SemaphoreType.DMA,
            pltpu.SemaphoreType.DMA,
            pltpu.SemaphoreType.DMA,
            pltpu.SemaphoreType.DMA,
            pltpu.SemaphoreType.DMA,
            pltpu.SemaphoreType.DMA,
            pltpu.SemaphoreType.DMA,
        ],
        compiler_params=pltpu.CompilerParams(use_tc_tiling_on_sc=False),
    )
    def body(ids_hbm, tgt_hbm, table_hbm, out_hbm, idx_v, idx2_v, tgt_v, buf,
             zeros_v, acc_s, *sems):
        sem_g, sem_s = sems[:_NSLOTS], sems[_NSLOTS:]
        sid = lax.axis_index("s")
        wid = sid * nc + lax.axis_index("c")
        pltpu.sync_copy(ids_hbm.at[pl.ds(wid * ipw, ipw)], idx_v)
        pltpu.sync_copy(tgt_hbm.at[sid], tgt_v)

        def dbl(i, c):
            for o in (0, 16, 32, 34):
                idx2_v[i, pl.ds(o, _LANES)] = (
                    idx_v[i, pl.ds(o, _LANES)] * 2)
            return c

        lax.fori_loop(0, ipw, dbl, 0)

        zeros = jnp.zeros((_LANES,), jnp.float32)

        def zbody(i, c):
            for j in range(nvec):
                zeros_v[i, pl.ds(_LANES * j, _LANES)] = zeros
            return c

        lax.fori_loop(0, ipw, zbody, 0)
        pltpu.sync_copy(zeros_v, acc_s.at[pl.ds(sid * ipw, ipw)])

        for c in range(_LEAD):
            pltpu.async_copy(table_hbm.at[idx2_v.at[c]], buf.at[c], sem_g[c])

        def steps(kk, carry):
            for b in range(_NSLOTS):
                k = kk * _NSLOTS + b
                pltpu.make_async_copy(
                    table_hbm.at[idx2_v.at[k]], buf.at[b], sem_g[b]).wait()
                pltpu.async_copy(
                    buf.at[b], acc_s.at[tgt_v.at[k]], sem_s[b], add=True)
                g = k + _LEAD
                bg = (b + _LEAD) % _NSLOTS

                @pl.when(g < ipw)
                def _():
                    @pl.when(g >= _NSLOTS)
                    def _():
                        pltpu.make_async_copy(
                            buf.at[bg], acc_s.at[tgt_v.at[k]], sem_s[bg]).wait()

                    pltpu.async_copy(
                        table_hbm.at[idx2_v.at[g]], buf.at[bg], sem_g[bg])
            return carry

        lax.fori_loop(0, ipw // _NSLOTS, steps, 0)

        for b in range(_NSLOTS):
            pltpu.make_async_copy(
                buf.at[b], acc_s.at[tgt_v.at[0]], sem_s[b]).wait()

        pltpu.sync_copy(acc_s.at[pl.ds(sid * ipw, ipw)],
                        out_hbm.at[pl.ds(wid * ipw, ipw)])

    return body(ids, tgt, table)


def _mm_body(scale, r_ref, t_ref, o_ref):
    o_ref[...] = jnp.dot(
        r_ref[...], t_ref[...], preferred_element_type=jnp.float32) * scale


def _matmul_tc(ratio, sums, scale):
    b, n = ratio.shape
    _, d = sums.shape
    bb = 256
    return pl.pallas_call(
        functools.partial(_mm_body, scale),
        grid=(b // bb,),
        in_specs=[
            pl.BlockSpec((bb, n), lambda i: (i, 0)),
            pl.BlockSpec((n, d), lambda i: (0, 0)),
        ],
        out_specs=pl.BlockSpec((bb, d), lambda i: (i, 0)),
        out_shape=jax.ShapeDtypeStruct((b, d), jnp.float32),
    )(ratio, sums)


def kernel(input_ids, input_ratio, embedding):
    n_items, hist = input_ids.shape
    vocab, d = embedding.shape
    info = plsc.get_sparse_core_info()
    ns = info.num_subcores
    ipw = n_items // (info.num_cores * ns)
    table_rm = _detile_tc(embedding.T).reshape(2 * vocab, d)
    tgt = jnp.broadcast_to(
        (jnp.arange(ns, dtype=jnp.int32)[:, None] * ipw
         + jnp.arange(ipw, dtype=jnp.int32)[None, :])[:, :, None],
        (ns, ipw, hist))
    sums = _gather_sum_sc(input_ids.astype(jnp.int32), tgt, table_rm)
    return _matmul_tc(input_ratio, sums, float(1.0 / hist))


# speedup vs baseline: 4.6605x; 1.0245x over previous
import functools

import jax
import jax.numpy as jnp
from jax import lax
from jax.experimental import pallas as pl
from jax.experimental.pallas import tpu as pltpu
from jax.experimental.pallas import tpu_sc as plsc

_LANES = 16
_NSLOTS = 4
_LEAD = 2


def _detile_body(bcols, x_ref, o_ref):
    h = bcols // 2
    x = x_ref[...]
    w = jnp.concatenate([x[:, :h].T, x[:, h:].T], axis=1)
    o_ref[...] = jnp.reshape(w, (bcols * x_ref.shape[0],))


def _detile_tc(table_t):
    d, v = table_t.shape
    bcols = 8192
    return pl.pallas_call(
        functools.partial(_detile_body, bcols),
        grid=(pl.cdiv(v, bcols),),
        in_specs=[pl.BlockSpec((d, bcols), lambda i: (0, i))],
        out_specs=pl.BlockSpec((bcols * d,), lambda i: (i,)),
        out_shape=jax.ShapeDtypeStruct((pl.cdiv(v, bcols) * bcols * d,),
                                       jnp.float32),
    )(table_t)


def _gather_sum_sc(ids, tgt, table):
    n_items, hist = ids.shape
    _, d = table.shape
    info = plsc.get_sparse_core_info()
    nc, ns = info.num_cores, info.num_subcores
    nw = nc * ns
    ipw = n_items // nw
    nvec = d // _LANES
    mesh = plsc.VectorSubcoreMesh(core_axis_name="c", subcore_axis_name="s")

    @functools.partial(
        pl.kernel,
        out_type=jax.ShapeDtypeStruct((n_items, d), jnp.float32),
        mesh=mesh,
        scratch_types=[
            pltpu.VMEM((ipw, hist), jnp.int32),
            pltpu.VMEM((ipw, hist), jnp.int32),
            pltpu.VMEM((ipw, hist), jnp.int32),
            pltpu.VMEM((_NSLOTS, hist, d), jnp.float32),
            pltpu.VMEM((ipw, d), jnp.float32),
            pltpu.VMEM_SHARED((ns * ipw, d), jnp.float32),
            pltpu.SemaphoreType.DMA,
            pltpu.SemaphoreType.DMA,
            pltpu.SemaphoreType.DMA,
            pltpu.SemaphoreType.DMA,
            pltpu.SemaphoreType.DMA,
            pltpu.SemaphoreType.DMA,
            pltpu.SemaphoreType.DMA,
            pltpu.SemaphoreType.DMA,
        ],
        compiler_params=pltpu.CompilerParams(use_tc_tiling_on_sc=False),
    )
    def body(ids_hbm, tgt_hbm, table_hbm, out_hbm, idx_v, idx2_v, tgt_v, buf,
             zeros_v, acc_s, *sems):
        sem_g, sem_s = sems[:_NSLOTS], sems[_NSLOTS:]
        sid = lax.axis_index("s")
        wid = sid * nc + lax.axis_index("c")
        pltpu.sync_copy(ids_hbm.at[pl.ds(wid * ipw, ipw)], idx_v)
        pltpu.sync_copy(tgt_hbm.at[sid], tgt_v)

        def perm(i, c):
            for o in (0, 16, 32, 34):
                raw = idx_v[i, pl.ds(o, _LANES)]
                u = lax.bitwise_and(raw, 8191)
                fr = (lax.bitwise_and(raw, -8192)
                      + lax.bitwise_and(u * 2, 8191)
                      + lax.shift_right_logical(u, 12))
                idx2_v[i, pl.ds(o, _LANES)] = fr
            return c

        lax.fori_loop(0, ipw, perm, 0)

        zeros = jnp.zeros((_LANES,), jnp.float32)

        def zbody(i, c):
            for j in range(nvec):
                zeros_v[i, pl.ds(_LANES * j, _LANES)] = zeros
            return c

        lax.fori_loop(0, ipw, zbody, 0)
        pltpu.sync_copy(zeros_v, acc_s.at[pl.ds(sid * ipw, ipw)])

        for c in range(_LEAD):
            pltpu.async_copy(table_hbm.at[idx2_v.at[c]], buf.at[c], sem_g[c])

        def steps(kk, carry):
            for b in range(_NSLOTS):
                k = kk * _NSLOTS + b
                pltpu.make_async_copy(
                    table_hbm.at[idx2_v.at[k]], buf.at[b], sem_g[b]).wait()
                pltpu.async_copy(
                    buf.at[b], acc_s.at[tgt_v.at[k]], sem_s[b], add=True)
                g = k + _LEAD
                bg = (b + _LEAD) % _NSLOTS

                @pl.when(g < ipw)
                def _():
                    @pl.when(g >= _NSLOTS)
                    def _():
                        pltpu.make_async_copy(
                            buf.at[bg], acc_s.at[tgt_v.at[k]], sem_s[bg]).wait()

                    pltpu.async_copy(
                        table_hbm.at[idx2_v.at[g]], buf.at[bg], sem_g[bg])
            return carry

        lax.fori_loop(0, ipw // _NSLOTS, steps, 0)

        for b in range(_NSLOTS):
            pltpu.make_async_copy(
                buf.at[b], acc_s.at[tgt_v.at[0]], sem_s[b]).wait()

        pltpu.sync_copy(acc_s.at[pl.ds(sid * ipw, ipw)],
                        out_hbm.at[pl.ds(wid * ipw, ipw)])

    return body(ids, tgt, table)


def _mm_body(scale, r_ref, t_ref, o_ref):
    o_ref[...] = jnp.dot(
        r_ref[...], t_ref[...], preferred_element_type=jnp.float32) * scale


def _matmul_tc(ratio, sums, scale):
    b, n = ratio.shape
    _, d = sums.shape
    bb = 256
    return pl.pallas_call(
        functools.partial(_mm_body, scale),
        grid=(b // bb,),
        in_specs=[
            pl.BlockSpec((bb, n), lambda i: (i, 0)),
            pl.BlockSpec((n, d), lambda i: (0, 0)),
        ],
        out_specs=pl.BlockSpec((bb, d), lambda i: (i, 0)),
        out_shape=jax.ShapeDtypeStruct((b, d), jnp.float32),
    )(ratio, sums)


def kernel(input_ids, input_ratio, embedding):
    n_items, hist = input_ids.shape
    vocab, d = embedding.shape
    info = plsc.get_sparse_core_info()
    ns = info.num_subcores
    ipw = n_items // (info.num_cores * ns)
    flat = _detile_tc(embedding.T)
    table_rm = flat.reshape(flat.shape[0] // d, d)
    tgt = jnp.broadcast_to(
        (jnp.arange(ns, dtype=jnp.int32)[:, None] * ipw
         + jnp.arange(ipw, dtype=jnp.int32)[None, :])[:, :, None],
        (ns, ipw, hist))
    sums = _gather_sum_sc(input_ids.astype(jnp.int32), tgt, table_rm)
    return _matmul_tc(input_ratio, sums, float(1.0 / hist))
